# baseline (speedup 1.0000x reference)
import functools

import jax
import jax.numpy as jnp
from jax import lax
from jax.experimental import pallas as pl
from jax.experimental.pallas import tpu as pltpu
from jax.experimental.pallas import tpu_sc as plsc

B = 16384
EDIM = 64
ALPHA = 0.001
BETA = 0.001
L2RG = 0.0001

NC = 2
NS = 16
NW = NC * NS
RPW = B // NW
CHUNK = 128
NCH = RPW // CHUNK

_sc_mesh = plsc.VectorSubcoreMesh(core_axis_name="c", subcore_axis_name="s")


@functools.partial(
    pl.kernel,
    out_type=[jax.ShapeDtypeStruct((B, EDIM), jnp.float32) for _ in range(4)]
    + [jax.ShapeDtypeStruct((CHUNK, EDIM), jnp.float32)],
    mesh=_sc_mesh,
    scratch_types=[
        pltpu.VMEM((RPW,), jnp.int32),
        pltpu.VMEM((RPW,), jnp.int32),
        pltpu.VMEM((CHUNK, EDIM), jnp.float32),
        pltpu.VMEM((CHUNK, EDIM), jnp.float32),
        pltpu.SemaphoreType.DMA,
        pltpu.SemaphoreType.DMA,
    ],
)
def _sc_gather(u_idx, nu_idx, p_idx, n_idx, ut, it,
               u_out, nu_out, p_out, n_out, dummy_out,
               idx_va, idx_vb, rows_a, rows_b, sem_a, sem_b):
    wid = lax.axis_index("s") * NC + lax.axis_index("c")
    base = wid * RPW

    def fire_rows(tab, idx_v, rows, sem, c):
        def _fire(je, _):
            rv = idx_v[pl.ds(c * CHUNK + je * 16, 16)]
            for dd in range(16):
                r = rv[dd]
                pltpu.async_copy(tab.at[pl.ds(r, 1)],
                                 rows.at[pl.ds(je * 16 + dd, 1)], sem)
            return 0
        lax.fori_loop(0, CHUNK // 16, _fire, 0)

    def drain_rows(rows, sem):
        pltpu.make_async_copy(dummy_out, rows, sem).wait()

    for idx_hbm, tab, out, idx_v, rows, sem in (
            (u_idx, ut, u_out, idx_va, rows_a, sem_a),
            (p_idx, it, p_out, idx_vb, rows_b, sem_b),
            (nu_idx, ut, nu_out, idx_va, rows_a, sem_a),
            (n_idx, it, n_out, idx_vb, rows_b, sem_b)):
        pltpu.sync_copy(idx_hbm.at[pl.ds(base, RPW)], idx_v)
        for c in range(NCH):
            fire_rows(tab, idx_v, rows, sem, c)
            drain_rows(rows, sem)
            pltpu.sync_copy(rows, out.at[pl.ds(base + c * CHUNK, CHUNK)])


def _tc_loss_body(u_ref, nu_ref, p_ref, n_ref, rate_ref,
                  uw_ref, ub_ref, iw_ref, ib_ref, out_ref):
    u = u_ref[...]
    nu = nu_ref[...]
    p = p_ref[...]
    n = n_ref[...]
    uw = uw_ref[...].reshape(1, EDIM)
    iw = iw_ref[...].reshape(1, EDIM)
    ub = ub_ref[0, 0]
    ib = ib_ref[0, 0]

    pu = jnp.sum(u * uw, axis=1, keepdims=True) + ub
    nu_l = jnp.sum(nu * uw, axis=1, keepdims=True) + ub
    pi = jnp.sum(p * iw, axis=1, keepdims=True) + ib
    ni = jnp.sum(n * iw, axis=1, keepdims=True) + ib
    dot = jnp.sum(u * p, axis=1, keepdims=True)

    pred = 1.0 + 4.0 * jax.nn.sigmoid(jax.nn.sigmoid(pu) * jax.nn.sigmoid(pi) * dot)
    rate_loss = jnp.mean((pred - rate_ref[...]) ** 2)
    user_loss = jnp.mean(jax.nn.softplus(-pu)) + jnp.mean(jax.nn.softplus(nu_l))
    item_loss = jnp.mean(jax.nn.softplus(-pi)) + jnp.mean(jax.nn.softplus(ni))
    reg = (jnp.sum(u * u) + jnp.sum(p * p) + jnp.sum(n * n)) * (0.5 / B)
    loss = rate_loss + ALPHA * user_loss + BETA * item_loss + L2RG * reg
    out_ref[...] = loss.reshape(1, 1)


_tc_loss = pl.pallas_call(
    _tc_loss_body,
    out_shape=jax.ShapeDtypeStruct((1, 1), jnp.float32),
)


def kernel(user, u_ir, nbr, item, rate, neg_user, neg_item,
           user_table, item_table, user_w, user_b, item_w, item_b):
    del u_ir, nbr
    u_emb, nu_emb, p_emb, n_emb = _sc_gather(
        user.astype(jnp.int32), neg_user.astype(jnp.int32),
        item.astype(jnp.int32), neg_item.astype(jnp.int32),
        user_table, item_table)[:4]
    loss = _tc_loss(u_emb, nu_emb, p_emb, n_emb, rate.reshape(B, 1),
                    user_w, user_b.reshape(1, 1), item_w, item_b.reshape(1, 1))
    return loss.reshape(())
